# Initial kernel scaffold; baseline (speedup 1.0000x reference)
import jax
import jax.numpy as jnp
from jax.experimental import pallas as pl


def kernel(query, reference_points, input_flatten, input_spatial_shapes, input_level_start_index, W_offsets, b_offsets, W_attn, b_attn, W_value, b_value, W_out, b_out):
    raise NotImplementedError("write your pallas kernel here")


# speedup vs baseline: 124.9274x; 124.9274x over previous
import functools

import jax
import jax.numpy as jnp
import numpy as np
from jax import lax
from jax.experimental import pallas as pl
from jax.experimental.pallas import tpu as pltpu
from jax.experimental.pallas import tpu_sc as plsc

SPATIAL_SHAPES = [(92, 92), (46, 46), (23, 23), (12, 12)]
D_MODEL = 256
N_HEADS = 8
N_LEVELS = 4
N_POINTS = 4
D_HEAD = 32
LEN_IN = 11253
B = 2
Q_TILE = 512
N_TILES = 22
LEN_PAD = Q_TILE * N_TILES
QT = B * LEN_PAD
N_WORKERS = 32
QPW = QT // N_WORKERS
MACRO = 16
N_MACRO = QPW // MACRO
SUB = 2
N_SUB = MACRO // SUB
ROWS_PER_B = LEN_PAD * N_HEADS


def _col_consts():
    j = np.arange(128)
    l = (j >> 2) & 3
    h = j >> 4
    wv = np.array([w for (_, w) in SPATIAL_SHAPES], np.float32)
    hv = np.array([hh for (hh, _) in SPATIAL_SHAPES], np.float32)
    starts = np.concatenate([[0], np.cumsum([hh * w for hh, w in SPATIAL_SHAPES])[:-1]])
    wc = wv[l]
    hc = hv[l]
    cc = (starts[l] * 8 + h).astype(np.int32)
    return (wc.reshape(1, 128), hc.reshape(1, 128),
            wc.astype(np.int32).reshape(1, 128), cc.reshape(1, 128))


def _value_body(x_ref, w_ref, o_ref):
    o_ref[...] = jnp.dot(x_ref[0], w_ref[...],
                         preferred_element_type=jnp.float32)[None]


def _outproj_body(x_ref, w_ref, o_ref):
    o_ref[...] = jnp.dot(x_ref[0], w_ref[...],
                         preferred_element_type=jnp.float32)[None]


def _prep_body(q_ref, rx_ref, ry_ref, wox_ref, woy_ref, wat_ref, bd_ref,
               wcf_ref, hcf_ref, wci_ref, cc_ref,
               i0_ref, i1_ref, i2_ref, i3_ref,
               w0_ref, w1_ref, w2_ref, w3_ref):
    bi = pl.program_id(0)
    q = q_ref[0]
    xo = jnp.dot(q, wox_ref[...], preferred_element_type=jnp.float32)
    yo = jnp.dot(q, woy_ref[...], preferred_element_type=jnp.float32)
    a = jnp.dot(q, wat_ref[...], preferred_element_type=jnp.float32)
    e = jnp.exp(a)
    denom = jnp.dot(e, bd_ref[...], preferred_element_type=jnp.float32)
    aw = e / denom

    wcf = wcf_ref[0]
    hcf = hcf_ref[0]
    x = rx_ref[0] * wcf + xo - 0.5
    y = ry_ref[0] * hcf + yo - 0.5
    x0 = jnp.floor(x)
    y0 = jnp.floor(y)
    wx1 = x - x0
    wx0 = 1.0 - wx1
    wy1 = y - y0
    wy0 = 1.0 - wy1
    x1 = x0 + 1.0
    y1 = y0 + 1.0
    vx0 = (x0 >= 0.0) & (x0 <= wcf - 1.0)
    vx1 = (x1 >= 0.0) & (x1 <= wcf - 1.0)
    vy0 = (y0 >= 0.0) & (y0 <= hcf - 1.0)
    vy1 = (y1 >= 0.0) & (y1 <= hcf - 1.0)
    px0 = jnp.clip(x0, 0.0, wcf - 1.0).astype(jnp.int32)
    px1 = jnp.clip(x1, 0.0, wcf - 1.0).astype(jnp.int32)
    py0 = jnp.clip(y0, 0.0, hcf - 1.0).astype(jnp.int32)
    py1 = jnp.clip(y1, 0.0, hcf - 1.0).astype(jnp.int32)

    wci = wci_ref[0]
    base = cc_ref[0] + bi * ROWS_PER_B
    i0_ref[0] = (py0 * wci + px0) * 8 + base
    i1_ref[0] = (py0 * wci + px1) * 8 + base
    i2_ref[0] = (py1 * wci + px0) * 8 + base
    i3_ref[0] = (py1 * wci + px1) * 8 + base
    zero = jnp.zeros_like(aw)
    w0_ref[0] = jnp.where(vx0 & vy0, wx0 * wy0 * aw, zero)
    w1_ref[0] = jnp.where(vx1 & vy0, wx1 * wy0 * aw, zero)
    w2_ref[0] = jnp.where(vx0 & vy1, wx0 * wy1 * aw, zero)
    w3_ref[0] = jnp.where(vx1 & vy1, wx1 * wy1 * aw, zero)


def _sc_body(value_hbm, i0, i1, i2, i3, w0, w1, w2, w3, out_hbm,
             idx_m, w_m, rows, out_v, csem, gsem):
    wid = lax.axis_index("s") * 2 + lax.axis_index("c")
    q0w = wid * QPW
    iarrs = (i0, i1, i2, i3)
    warrs = (w0, w1, w2, w3)

    def macro_body(m, carry):
        qb = q0w + m * MACRO
        descs = []
        for c in range(4):
            descs.append(pltpu.async_copy(
                iarrs[c].at[pl.ds(qb, MACRO)],
                idx_m.at[pl.ds(c * MACRO, MACRO)], csem))
            descs.append(pltpu.async_copy(
                warrs[c].at[pl.ds(qb, MACRO)],
                w_m.at[pl.ds(c * MACRO, MACRO)], csem))
        for d in descs:
            d.wait()

        def sub_body(s, carry2):
            qs0 = s * SUB
            gds = []
            for c in range(4):
                for qq in range(SUB):
                    gds.append(pltpu.async_copy(
                        value_hbm.at[idx_m.at[c * MACRO + qs0 + qq]],
                        rows.at[c * SUB + qq], gsem))
            for d in gds:
                d.wait()

            def qh_body(qh, carry3):
                qs = qh >> 3
                h = qh & 7
                acc0 = jnp.zeros((16,), jnp.float32)
                acc1 = jnp.zeros((16,), jnp.float32)
                h16 = h * 16
                for c in range(4):
                    r = c * SUB + qs
                    wr = c * MACRO + qs0 + qs
                    w16 = w_m[wr, pl.ds(h16, 16)]
                    for j in range(16):
                        wv = w16[j]
                        acc0 = acc0 + wv * rows[r, h16 + j, pl.ds(0, 16)]
                        acc1 = acc1 + wv * rows[r, h16 + j, pl.ds(16, 16)]
                out_v[qs0 + qs, pl.ds(h * 32, 16)] = acc0
                out_v[qs0 + qs, pl.ds(h * 32 + 16, 16)] = acc1
                return carry3

            lax.fori_loop(0, SUB * N_HEADS, qh_body, 0, unroll=False)
            return carry2

        lax.fori_loop(0, N_SUB, sub_body, 0, unroll=False)
        pltpu.sync_copy(out_v, out_hbm.at[pl.ds(qb, MACRO)])
        return carry

    lax.fori_loop(0, N_MACRO, macro_body, 0, unroll=False)


def kernel(query, reference_points, input_flatten, input_spatial_shapes,
           input_level_start_index, W_offsets, b_offsets, W_attn, b_attn,
           W_value, b_value, W_out, b_out):
    f32 = jnp.float32
    pad_q = LEN_PAD - LEN_IN

    query_p = jnp.pad(query, ((0, 0), (0, pad_q), (0, 0)))
    input_p = jnp.pad(input_flatten, ((0, 0), (0, pad_q), (0, 0)))
    refx = jnp.tile(jnp.repeat(reference_points[..., 0], 4, axis=2), (1, 1, 8))
    refy = jnp.tile(jnp.repeat(reference_points[..., 1], 4, axis=2), (1, 1, 8))
    refx = jnp.pad(refx, ((0, 0), (0, pad_q), (0, 0)))
    refy = jnp.pad(refy, ((0, 0), (0, pad_q), (0, 0)))

    w_off = W_offsets.reshape(D_MODEL, 128, 2)
    w_off_x = w_off[:, :, 0]
    w_off_y = w_off[:, :, 1]
    bd = jnp.asarray(np.kron(np.eye(8, dtype=np.float32), np.ones((16, 16), np.float32)))
    wcf, hcf, wci, cc = _col_consts()
    wcf = jnp.asarray(wcf)
    hcf = jnp.asarray(hcf)
    wci = jnp.asarray(wci)
    cc = jnp.asarray(cc)

    qspec = pl.BlockSpec((1, Q_TILE, D_MODEL), lambda b, t: (b, t, 0))
    cspec128 = pl.BlockSpec((1, Q_TILE, 128), lambda b, t: (b, t, 0))
    wspec = pl.BlockSpec((D_MODEL, D_MODEL), lambda b, t: (0, 0))
    wspec128 = pl.BlockSpec((D_MODEL, 128), lambda b, t: (0, 0))
    bdspec = pl.BlockSpec((128, 128), lambda b, t: (0, 0))
    constspec = pl.BlockSpec((1, 128), lambda b, t: (0, 0))

    value_pad = pl.pallas_call(
        _value_body,
        grid=(B, N_TILES),
        in_specs=[qspec, wspec],
        out_specs=qspec,
        out_shape=jax.ShapeDtypeStruct((B, LEN_PAD, D_MODEL), f32),
    )(input_p, W_value)
    value_tbl = value_pad.reshape(B * ROWS_PER_B, D_HEAD)

    outs = pl.pallas_call(
        _prep_body,
        grid=(B, N_TILES),
        in_specs=[qspec, cspec128, cspec128, wspec128, wspec128, wspec128,
                  bdspec, constspec, constspec, constspec, constspec],
        out_specs=[cspec128] * 8,
        out_shape=[jax.ShapeDtypeStruct((B, LEN_PAD, 128), jnp.int32)] * 4
                  + [jax.ShapeDtypeStruct((B, LEN_PAD, 128), f32)] * 4,
    )(query_p, refx, refy, w_off_x, w_off_y, W_attn, bd, wcf, hcf, wci, cc)
    i0, i1, i2, i3 = (o.reshape(QT, 128) for o in outs[:4])
    w0, w1, w2, w3 = (o.reshape(QT, 128) for o in outs[4:])

    sc_fn = pl.kernel(
        _sc_body,
        out_type=jax.ShapeDtypeStruct((QT, D_MODEL), f32),
        mesh=plsc.VectorSubcoreMesh(core_axis_name="c", subcore_axis_name="s"),
        compiler_params=pltpu.CompilerParams(use_tc_tiling_on_sc=False),
        scratch_types=[
            pltpu.VMEM((4 * MACRO, 128), jnp.int32),
            pltpu.VMEM((4 * MACRO, 128), f32),
            pltpu.VMEM((4 * SUB, 128, D_HEAD), f32),
            pltpu.VMEM((MACRO, D_MODEL), f32),
            pltpu.SemaphoreType.DMA,
            pltpu.SemaphoreType.DMA,
        ],
    )
    out_sc = sc_fn(value_tbl, i0, i1, i2, i3, w0, w1, w2, w3)

    out_proj = pl.pallas_call(
        _outproj_body,
        grid=(B, N_TILES),
        in_specs=[qspec, wspec],
        out_specs=qspec,
        out_shape=jax.ShapeDtypeStruct((B, LEN_PAD, D_MODEL), f32),
    )(out_sc.reshape(B, LEN_PAD, D_MODEL), W_out)
    return out_proj[:, :LEN_IN, :]
